# Initial kernel scaffold; baseline (speedup 1.0000x reference)
import jax
import jax.numpy as jnp
from jax.experimental import pallas as pl


def kernel(depth, features, guide_weight):
    raise NotImplementedError("write your pallas kernel here")


# speedup vs baseline: 33.9756x; 33.9756x over previous
import numpy as np
import jax
import jax.numpy as jnp
from jax.experimental import pallas as pl
from jax.experimental.pallas import tpu as pltpu

_K = 5
_DEPTH_MAX = 192.0
_SAMPLE_NUM = 15
_PAD = 2
_H = 224
_W = 224
_C = 96
_TH = 16


def _select_index():
    rng = np.random.default_rng(0)
    points = rng.choice(_K * _K, _SAMPLE_NUM, replace=True)
    rng.shuffle(points)
    cx = _K // 2
    cy = _K // 2
    px = points % _K
    py = points // _K
    dis = np.sqrt((px - cx) ** 2 + (py - cy) ** 2)
    w = np.exp(-0.5 * dis)
    w = w / np.sum(w)
    return points.astype(np.int32), w.astype(np.float32)


_PTS, _WTS = _select_index()
_PY = [int(p) // _K for p in _PTS]
_PX = [int(p) % _K for p in _PTS]
_WL = [float(w) for w in _WTS]
_PTS_L = [int(p) for p in _PTS]


def _pool_kernel(dpad_ref, guide_ref, feat_ref, out_ref):
    i = pl.program_id(0)
    r0 = i * _TH

    drows = dpad_ref[pl.ds(r0, _TH + 2 * _PAD), :]
    valid = jnp.where((drows > 0.0) & (drows < _DEPTH_MAX), 1.0, 0.0)

    logits = []
    for s in range(_SAMPLE_NUM):
        vs = valid[_PY[s]:_PY[s] + _TH, _PX[s]:_PX[s] + _W]
        g = guide_ref[_PTS_L[s]]
        logits.append(vs * (_WL[s] * g))
    lg = jnp.stack(logits, axis=0)
    m = jnp.max(lg, axis=0, keepdims=True)
    e = jnp.exp(lg - m)
    wn = e / jnp.sum(e, axis=0, keepdims=True)

    frows = feat_ref[:, pl.ds(r0, _TH + 2 * _PAD), :]
    acc = None
    for s in range(_SAMPLE_NUM):
        f = frows[:, _PY[s]:_PY[s] + _TH, _PX[s]:_PX[s] + _W]
        term = f * wn[s][None]
        acc = term if acc is None else acc + term
    out_ref[:] = acc


def kernel(depth, features, guide_weight):
    feat = features[0]
    fpad = jnp.pad(feat, ((0, 0), (_PAD, _PAD), (_PAD, _PAD)))
    dpad = jnp.pad(depth[0, 0], ((_PAD, _PAD), (_PAD, _PAD)))
    gt = jnp.transpose(guide_weight[0], (2, 0, 1))

    grid = (_H // _TH,)
    out = pl.pallas_call(
        _pool_kernel,
        grid=grid,
        in_specs=[
            pl.BlockSpec((_H + 2 * _PAD, _W + 2 * _PAD), lambda i: (0, 0)),
            pl.BlockSpec((_K * _K, _TH, _W), lambda i: (0, i, 0)),
            pl.BlockSpec((_C, _H + 2 * _PAD, _W + 2 * _PAD), lambda i: (0, 0, 0)),
        ],
        out_specs=pl.BlockSpec((_C, _TH, _W), lambda i: (0, i, 0)),
        out_shape=jax.ShapeDtypeStruct((_C, _H, _W), jnp.float32),
    )(dpad, gt, fpad)

    return out[None], features


# speedup vs baseline: 37.7461x; 1.1110x over previous
import numpy as np
import jax
import jax.numpy as jnp
from jax.experimental import pallas as pl
from jax.experimental.pallas import tpu as pltpu

_K = 5
_DEPTH_MAX = 192.0
_SAMPLE_NUM = 15
_PAD = 2
_H = 224
_W = 224
_C = 96
_TH = 16
_NT = _H // _TH
_SR = _TH + 16
_ROFF = 8 - _PAD


def _select_index():
    rng = np.random.default_rng(0)
    points = rng.choice(_K * _K, _SAMPLE_NUM, replace=True)
    rng.shuffle(points)
    cx = _K // 2
    cy = _K // 2
    px = points % _K
    py = points // _K
    dis = np.sqrt((px - cx) ** 2 + (py - cy) ** 2)
    w = np.exp(-0.5 * dis)
    w = w / np.sum(w)
    return points.astype(np.int32), w.astype(np.float32)


_PTS, _WTS = _select_index()
_PY = [int(p) // _K for p in _PTS]
_PX = [int(p) % _K for p in _PTS]
_WL = [float(w) for w in _WTS]
_PTS_L = [int(p) for p in _PTS]

_GROUPS = {}
for _s, _p in enumerate(_PTS_L):
    _GROUPS.setdefault(_p % _K, {}).setdefault(_p // _K, []).append(_s)
_GROUPS = {b: sorted(d.items()) for b, d in sorted(_GROUPS.items())}


def _pool_kernel(d_ref, gt_ref, f_ref, out_ref, fbuf, dbuf):
    i = pl.program_id(0)
    r0 = i * _TH
    wp = _W + 2 * _PAD

    @pl.when(i == 0)
    def _init_first():
        fbuf[:, :, 0:_PAD] = jnp.zeros((_C, _SR, _PAD), jnp.float32)
        fbuf[:, :, _PAD + _W:] = jnp.zeros((_C, _SR, _PAD), jnp.float32)
        dbuf[:, 0:_PAD] = jnp.zeros((_SR, _PAD), jnp.float32)
        dbuf[:, _PAD + _W:] = jnp.zeros((_SR, _PAD), jnp.float32)
        fbuf[:, 0:8, :] = jnp.zeros((_C, 8, wp), jnp.float32)
        dbuf[0:8, :] = jnp.zeros((8, wp), jnp.float32)
        fbuf[:, 8:_SR, _PAD:_PAD + _W] = f_ref[:, 0:_SR - 8, :]
        dbuf[8:_SR, _PAD:_PAD + _W] = d_ref[0:_SR - 8, :]

    @pl.when(i == _NT - 1)
    def _init_last():
        fbuf[:, _SR - 8:, :] = jnp.zeros((_C, 8, wp), jnp.float32)
        dbuf[_SR - 8:, :] = jnp.zeros((8, wp), jnp.float32)
        fbuf[:, 0:_SR - 8, _PAD:_PAD + _W] = f_ref[:, _H - (_SR - 8):_H, :]
        dbuf[0:_SR - 8, _PAD:_PAD + _W] = d_ref[_H - (_SR - 8):_H, :]

    @pl.when((i > 0) & (i < _NT - 1))
    def _init_mid():
        fbuf[:, :, _PAD:_PAD + _W] = f_ref[:, pl.ds(r0 - 8, _SR), :]
        dbuf[:, _PAD:_PAD + _W] = d_ref[pl.ds(r0 - 8, _SR), :]

    dv = dbuf[:, :]
    valid = jnp.where((dv > 0.0) & (dv < _DEPTH_MAX), 1.0, 0.0)

    logits = []
    for s in range(_SAMPLE_NUM):
        a = _ROFF + _PY[s]
        vs = valid[a:a + _TH, _PX[s]:_PX[s] + _W]
        g = gt_ref[_PTS_L[s]]
        logits.append(vs * (_WL[s] * g))
    lg = jnp.stack(logits, axis=0)
    m = jnp.max(lg, axis=0, keepdims=True)
    e = jnp.exp(lg - m)
    wn = e / jnp.sum(e, axis=0, keepdims=True)

    acc = None
    for b, rows in _GROUPS.items():
        cb = fbuf[:, _ROFF:_ROFF + _TH + 2 * _PAD, b:b + _W]
        for a, samples in rows:
            wm = wn[samples[0]]
            for s in samples[1:]:
                wm = wm + wn[s]
            term = cb[:, a:a + _TH, :] * wm[None]
            acc = term if acc is None else acc + term
    out_ref[:] = acc


def kernel(depth, features, guide_weight):
    feat = features[0]
    d = depth[0, 0]
    gt = jnp.transpose(guide_weight[0], (2, 0, 1))

    out = pl.pallas_call(
        _pool_kernel,
        grid=(_NT,),
        in_specs=[
            pl.BlockSpec((_H, _W), lambda i: (0, 0)),
            pl.BlockSpec((_K * _K, _TH, _W), lambda i: (0, i, 0)),
            pl.BlockSpec((_C, _H, _W), lambda i: (0, 0, 0)),
        ],
        out_specs=pl.BlockSpec((_C, _TH, _W), lambda i: (0, i, 0)),
        out_shape=jax.ShapeDtypeStruct((_C, _H, _W), jnp.float32),
        scratch_shapes=[
            pltpu.VMEM((_C, _SR, _W + 2 * _PAD), jnp.float32),
            pltpu.VMEM((_SR, _W + 2 * _PAD), jnp.float32),
        ],
    )(d, gt, feat)

    return out[None], features


# speedup vs baseline: 39.7950x; 1.0543x over previous
import numpy as np
import jax
import jax.numpy as jnp
from jax.experimental import pallas as pl
from jax.experimental.pallas import tpu as pltpu

_K = 5
_DEPTH_MAX = 192.0
_SAMPLE_NUM = 15
_PAD = 2
_H = 224
_W = 224
_C = 96
_TH = 16
_NT = _H // _TH
_SR = _TH + 16
_ROFF = 8 - _PAD


def _select_index():
    rng = np.random.default_rng(0)
    points = rng.choice(_K * _K, _SAMPLE_NUM, replace=True)
    rng.shuffle(points)
    cx = _K // 2
    cy = _K // 2
    px = points % _K
    py = points // _K
    dis = np.sqrt((px - cx) ** 2 + (py - cy) ** 2)
    w = np.exp(-0.5 * dis)
    w = w / np.sum(w)
    return points.astype(np.int32), w.astype(np.float32)


_PTS, _WTS = _select_index()
_PY = [int(p) // _K for p in _PTS]
_PX = [int(p) % _K for p in _PTS]
_WL = [float(w) for w in _WTS]
_PTS_L = [int(p) for p in _PTS]

_GROUPS = {}
for _s, _p in enumerate(_PTS_L):
    _GROUPS.setdefault(_p % _K, {}).setdefault(_p // _K, []).append(_s)
_GROUPS = {b: sorted(d.items()) for b, d in sorted(_GROUPS.items())}


def _pool_kernel(d_ref, gt_ref, f_ref, out_ref, fbuf, dbuf):
    i = pl.program_id(0)
    r0 = i * _TH
    wp = _W + 2 * _PAD

    @pl.when(i == 0)
    def _init_first():
        fbuf[:, :, 0:_PAD] = jnp.zeros((_C, _SR, _PAD), jnp.bfloat16)
        fbuf[:, :, _PAD + _W:] = jnp.zeros((_C, _SR, _PAD), jnp.bfloat16)
        dbuf[:, 0:_PAD] = jnp.zeros((_SR, _PAD), jnp.float32)
        dbuf[:, _PAD + _W:] = jnp.zeros((_SR, _PAD), jnp.float32)
        fbuf[:, 0:8, :] = jnp.zeros((_C, 8, wp), jnp.bfloat16)
        dbuf[0:8, :] = jnp.zeros((8, wp), jnp.float32)
        fbuf[:, 8:_SR, _PAD:_PAD + _W] = f_ref[:, 0:_SR - 8, :].astype(jnp.bfloat16)
        dbuf[8:_SR, _PAD:_PAD + _W] = d_ref[0:_SR - 8, :]

    @pl.when(i == _NT - 1)
    def _init_last():
        fbuf[:, _SR - 8:, :] = jnp.zeros((_C, 8, wp), jnp.bfloat16)
        dbuf[_SR - 8:, :] = jnp.zeros((8, wp), jnp.float32)
        fbuf[:, 0:_SR - 8, _PAD:_PAD + _W] = f_ref[:, _H - (_SR - 8):_H, :].astype(jnp.bfloat16)
        dbuf[0:_SR - 8, _PAD:_PAD + _W] = d_ref[_H - (_SR - 8):_H, :]

    @pl.when((i > 0) & (i < _NT - 1))
    def _init_mid():
        fbuf[:, :, _PAD:_PAD + _W] = f_ref[:, pl.ds(r0 - 8, _SR), :].astype(jnp.bfloat16)
        dbuf[:, _PAD:_PAD + _W] = d_ref[pl.ds(r0 - 8, _SR), :]

    dv = dbuf[:, :]
    valid = jnp.where((dv > 0.0) & (dv < _DEPTH_MAX), 1.0, 0.0)

    logits = []
    for s in range(_SAMPLE_NUM):
        a = _ROFF + _PY[s]
        vs = valid[a:a + _TH, _PX[s]:_PX[s] + _W]
        g = gt_ref[_PTS_L[s]]
        logits.append(vs * (_WL[s] * g))
    lg = jnp.stack(logits, axis=0)
    m = jnp.max(lg, axis=0, keepdims=True)
    e = jnp.exp(lg - m)
    wn = e / jnp.sum(e, axis=0, keepdims=True)

    acc = None
    for b, rows in _GROUPS.items():
        for a, samples in rows:
            wm = wn[samples[0]]
            for s in samples[1:]:
                wm = wm + wn[s]
            term = fbuf[:, _ROFF + a:_ROFF + a + _TH, b:b + _W].astype(jnp.float32) * wm[None]
            acc = term if acc is None else acc + term
    out_ref[:] = acc


def kernel(depth, features, guide_weight):
    feat = features[0]
    d = depth[0, 0]
    gt = jnp.transpose(guide_weight[0], (2, 0, 1))

    out = pl.pallas_call(
        _pool_kernel,
        grid=(_NT,),
        in_specs=[
            pl.BlockSpec((_H, _W), lambda i: (0, 0)),
            pl.BlockSpec((_K * _K, _TH, _W), lambda i: (0, i, 0)),
            pl.BlockSpec((_C, _H, _W), lambda i: (0, 0, 0)),
        ],
        out_specs=pl.BlockSpec((_C, _TH, _W), lambda i: (0, i, 0)),
        out_shape=jax.ShapeDtypeStruct((_C, _H, _W), jnp.float32),
        scratch_shapes=[
            pltpu.VMEM((_C, _SR, _W + 2 * _PAD), jnp.bfloat16),
            pltpu.VMEM((_SR, _W + 2 * _PAD), jnp.float32),
        ],
    )(d, gt, feat)

    return out[None], features
